# baseline (speedup 1.0000x reference)
import jax
import jax.numpy as jnp
from jax.experimental import pallas as pl
from jax.experimental.pallas import tpu as pltpu

_B, _CIN, _H, _W = 8, 384, 28, 28
_DIM = 256
_K = 1024
_HW = _H * _W
_PAD = 32
_PADROWS = _PAD + _HW + _PAD
_EPS = 1e-5

_BLOCKS = (
    (_CIN, _DIM, True),
    (_DIM, _DIM, False),
    (_DIM, _DIM, False),
    (_DIM, _DIM, False),
    (_DIM, _DIM, False),
    (_DIM, _CIN, True),
)


def _conv3x3(pad_ref, h, w_ref, cin):
    pad_ref[_PAD:_PAD + _HW, :cin] = h.astype(jnp.bfloat16)
    col = jax.lax.broadcasted_iota(jnp.int32, (_HW, 1), 0) % _W
    wins = []
    for dy in (-1, 0, 1):
        for dx in (-1, 0, 1):
            s = dy * _W + dx
            win = pad_ref[_PAD + s:_PAD + s + _HW, :cin]
            if dx != 0:
                ok = (col + dx >= 0) & (col + dx < _W)
                win = jnp.where(ok, win, jnp.bfloat16(0))
            wins.append(win)
    big = jnp.concatenate(wins, axis=-1)
    return jnp.dot(big, w_ref[...], preferred_element_type=jnp.float32)


def _bn(x, g_ref, b_ref):
    return x / jnp.sqrt(jnp.float32(1.0) + jnp.float32(_EPS)) * g_ref[...] \
        + b_ref[...]


def _resblock(pad_ref, x, refs, cin, cout, proj):
    g1, b1, w1, g2, b2, w2, c2b = refs[:7]
    h = jnp.maximum(_bn(x, g1, b1), 0.0)
    h = _conv3x3(pad_ref, h, w1, cin)
    h = jnp.maximum(_bn(h, g2, b2), 0.0)
    h = _conv3x3(pad_ref, h, w2, cin) + c2b[...]
    if proj:
        sg, sb, sw, swb = refs[7:11]
        idn = _bn(x, sg, sb).astype(jnp.bfloat16)
        idn = jnp.dot(idn, sw[...],
                      preferred_element_type=jnp.float32) + swb[...]
    else:
        idn = x
    return h + idn


def _pipeline_body(x_ref, *rest):
    pad_ref = rest[-1]
    recon_ref, ind_ref = rest[-3], rest[-2]
    refs = list(rest[:-3])

    pad_ref[0:_PAD, :] = jnp.zeros((_PAD, _CIN), jnp.bfloat16)
    pad_ref[_PAD + _HW:_PADROWS, :] = jnp.zeros((_PAD, _CIN), jnp.bfloat16)

    z = x_ref[0]
    pos = 0
    for bi, (cin, cout, proj) in enumerate(_BLOCKS):
        n = 11 if proj else 7
        z = _resblock(pad_ref, z, refs[pos:pos + n], cin, cout, proj)
        pos += n
        if bi == 2:
            s0, b0, embed_bf, embed_f32, embedT = refs[-5:]
            z = z * s0[...] + b0[...]
            score = jnp.dot(z.astype(jnp.bfloat16), embed_bf[...],
                            preferred_element_type=jnp.float32)
            e = embed_f32[...]
            e2 = jnp.sum(e * e, axis=0, keepdims=True)
            f2 = jnp.sum(z * z, axis=1, keepdims=True)
            dist = (f2 + e2) - 2.0 * score
            m = jnp.min(dist, axis=1, keepdims=True)
            iota_k = jax.lax.broadcasted_iota(jnp.int32, (_HW, _K), 1)
            ind = jnp.min(jnp.where(dist == m, iota_k, _K), axis=1)
            onehot = (iota_k == ind[:, None]).astype(jnp.float32)
            q = jnp.dot(onehot, embedT[...],
                        precision=jax.lax.Precision.HIGHEST,
                        preferred_element_type=jnp.float32)
            ind_ref[...] = ind.reshape(1, 1, _HW)
            z = z + (q - z)

    recon_ref[0] = z


def _prep_params(params):
    arrs = []
    for p in list(params['enc']) + list(params['dec']):
        ci = p['conv1_w'].shape[1]
        co = p['conv2_w'].shape[0]
        arrs.append(p['bn1_g'][None, :])
        arrs.append(p['bn1_b'][None, :])
        arrs.append(jnp.transpose(p['conv1_w'], (2, 3, 1, 0))
                    .reshape(9 * ci, ci).astype(jnp.bfloat16))
        arrs.append(p['bn2_g'][None, :])
        arrs.append(p['bn2_b'][None, :])
        arrs.append(jnp.transpose(p['conv2_w'], (2, 3, 1, 0))
                    .reshape(9 * ci, co).astype(jnp.bfloat16))
        arrs.append(p['conv2_b'][None, :])
        if 'conv_s_w' in p:
            arrs.append(p['bn_s_g'][None, :])
            arrs.append(p['bn_s_b'][None, :])
            arrs.append(jnp.transpose(p['conv_s_w'][:, :, 0, 0])
                        .astype(jnp.bfloat16))
            arrs.append(p['conv_s_b'][None, :])
    arrs.append(params['scale0'].reshape(1, _DIM))
    arrs.append(params['bias0'].reshape(1, _DIM))
    embed = params['vq1']
    arrs.append(embed.astype(jnp.bfloat16))
    arrs.append(embed)
    arrs.append(jnp.transpose(embed))
    return arrs


def kernel(feat, params):
    x = jnp.transpose(feat, (0, 2, 3, 1)).reshape(_B, _HW, _CIN)
    arrs = _prep_params(params)

    def _full_spec(a):
        nd = a.ndim
        return pl.BlockSpec(a.shape, lambda b, _nd=nd: (0,) * _nd)

    in_specs = [pl.BlockSpec((1, _HW, _CIN), lambda b: (b, 0, 0))]
    in_specs += [_full_spec(a) for a in arrs]

    recon, ind = pl.pallas_call(
        _pipeline_body,
        grid=(_B,),
        in_specs=in_specs,
        out_specs=[
            pl.BlockSpec((1, _HW, _CIN), lambda b: (b, 0, 0)),
            pl.BlockSpec((1, 1, _HW), lambda b: (b, 0, 0)),
        ],
        out_shape=[
            jax.ShapeDtypeStruct((_B, _HW, _CIN), jnp.float32),
            jax.ShapeDtypeStruct((_B, 1, _HW), jnp.int32),
        ],
        scratch_shapes=[pltpu.VMEM((_PADROWS, _CIN), jnp.bfloat16)],
        compiler_params=pltpu.CompilerParams(
            vmem_limit_bytes=120 * 1024 * 1024),
    )(x, *arrs)

    recon = jnp.transpose(recon.reshape(_B, _H, _W, _CIN), (0, 3, 1, 2))
    return recon, ind.reshape(_B, _H, _W)
